# baseline (speedup 1.0000x reference)
import jax
import jax.numpy as jnp
from jax.experimental import pallas as pl

N_UNITS_ = 1000
ROWS_BLK = 512


def _onehot_body(idx_ref, out_ref):
    iota = jax.lax.broadcasted_iota(jnp.int32, out_ref.shape, 1)
    out_ref[...] = (idx_ref[...] == iota).astype(jnp.float32)


def kernel(indices):
    batch, n_active = indices.shape
    rows = batch * n_active
    flat = indices.reshape(rows, 1)
    out = pl.pallas_call(
        _onehot_body,
        grid=(rows // ROWS_BLK,),
        in_specs=[pl.BlockSpec((ROWS_BLK, 1), lambda i: (i, 0))],
        out_specs=pl.BlockSpec((ROWS_BLK, N_UNITS_), lambda i: (i, 0)),
        out_shape=jax.ShapeDtypeStruct((rows, N_UNITS_), jnp.float32),
    )(flat)
    return out.reshape(batch, n_active, N_UNITS_)


# speedup vs baseline: 1.5246x; 1.5246x over previous
import jax
import jax.numpy as jnp
from jax.experimental import pallas as pl

N_UNITS_ = 1000
BATCH_BLK = 128


def _onehot_body(idx_ref, out_ref):
    iota = jax.lax.broadcasted_iota(jnp.int32, out_ref.shape, 2)
    out_ref[...] = (idx_ref[...][:, :, None] == iota).astype(jnp.float32)


def kernel(indices):
    batch, n_active = indices.shape
    return pl.pallas_call(
        _onehot_body,
        grid=(batch // BATCH_BLK,),
        in_specs=[pl.BlockSpec((BATCH_BLK, n_active), lambda i: (i, 0))],
        out_specs=pl.BlockSpec((BATCH_BLK, n_active, N_UNITS_), lambda i: (i, 0, 0)),
        out_shape=jax.ShapeDtypeStruct((batch, n_active, N_UNITS_), jnp.float32),
    )(indices)


# speedup vs baseline: 1.5374x; 1.0084x over previous
import jax
import jax.numpy as jnp
from jax.experimental import pallas as pl
from jax.experimental.pallas import tpu as pltpu

N_UNITS_ = 1000
BATCH_BLK = 64
NBUF = 4


def _onehot_body(idx_ref, out_hbm, scratch, sems):
    i = pl.program_id(0)
    nb = pl.num_programs(0)
    slot = jax.lax.rem(i, NBUF)

    def _copy(s, blk):
        return pltpu.make_async_copy(
            scratch.at[s],
            out_hbm.at[pl.ds(blk * BATCH_BLK, BATCH_BLK)],
            sems.at[s],
        )

    @pl.when(i >= NBUF)
    def _():
        _copy(slot, i - NBUF).wait()

    iota = jax.lax.broadcasted_iota(jnp.int32, (BATCH_BLK, idx_ref.shape[1], N_UNITS_), 2)
    scratch[slot] = (idx_ref[...][:, :, None] == iota).astype(jnp.float32)
    _copy(slot, i).start()

    @pl.when(i == nb - 1)
    def _():
        for k in range(NBUF):
            blk = nb - NBUF + k
            _copy(jax.lax.rem(blk, NBUF), blk).wait()


def kernel(indices):
    batch, n_active = indices.shape
    return pl.pallas_call(
        _onehot_body,
        grid=(batch // BATCH_BLK,),
        in_specs=[pl.BlockSpec((BATCH_BLK, n_active), lambda i: (i, 0))],
        out_specs=pl.BlockSpec(memory_space=pl.ANY),
        out_shape=jax.ShapeDtypeStruct((batch, n_active, N_UNITS_), jnp.float32),
        scratch_shapes=[
            pltpu.VMEM((NBUF, BATCH_BLK, n_active, N_UNITS_), jnp.float32),
            pltpu.SemaphoreType.DMA((NBUF,)),
        ],
    )(indices)


# speedup vs baseline: 6.9149x; 4.4977x over previous
import jax
import jax.numpy as jnp
from jax.experimental import pallas as pl

N_UNITS_ = 1000
A_BLK = 2


def _onehot_body(idxt_ref, out_ref):
    iota = jax.lax.broadcasted_iota(jnp.int32, out_ref.shape, 1)
    out_ref[...] = (idxt_ref[...][:, 0, :][:, None, :] == iota).astype(jnp.float32)


def kernel(indices):
    batch, n_active = indices.shape
    idx_t = indices.T.reshape(n_active, 1, batch)
    oh_t = pl.pallas_call(
        _onehot_body,
        grid=(n_active // A_BLK,),
        in_specs=[pl.BlockSpec((A_BLK, 1, batch), lambda i: (i, 0, 0))],
        out_specs=pl.BlockSpec((A_BLK, N_UNITS_, batch), lambda i: (i, 0, 0)),
        out_shape=jax.ShapeDtypeStruct((n_active, N_UNITS_, batch), jnp.float32),
    )(idx_t)
    return oh_t.transpose(2, 0, 1)


# speedup vs baseline: 7.5770x; 1.0957x over previous
import jax
import jax.numpy as jnp
from jax.experimental import pallas as pl

N_UNITS_ = 1000
U_BLK = 40


def _onehot_body(idxt_ref, out_ref):
    u0 = pl.program_id(0) * U_BLK
    iota = u0 + jax.lax.broadcasted_iota(jnp.int32, out_ref.shape, 1)
    out_ref[...] = (idxt_ref[...][:, None, :] == iota).astype(jnp.float32)


def kernel(indices):
    batch, n_active = indices.shape
    idx_t = indices.T
    oh_t = pl.pallas_call(
        _onehot_body,
        grid=(N_UNITS_ // U_BLK,),
        in_specs=[pl.BlockSpec((n_active, batch), lambda i: (0, 0))],
        out_specs=pl.BlockSpec((n_active, U_BLK, batch), lambda i: (0, i, 0)),
        out_shape=jax.ShapeDtypeStruct((n_active, N_UNITS_, batch), jnp.float32),
    )(idx_t)
    return oh_t.transpose(2, 0, 1)
